# Initial kernel scaffold; baseline (speedup 1.0000x reference)
import functools

import jax
import jax.numpy as jnp
from jax import lax
from jax.experimental import pallas as pl
from jax.experimental.pallas import tpu as pltpu
from jax.experimental.pallas import tpu_sc as plsc

N = 100000
E = 1600000
D_IN = 128
DIM = 32
HALF = 16
N_GRAPHS = 128
N_CLASSES = 10

BLK = 2000
GRID = N // BLK

EROWS = 12800
EPAD = EROWS * 128 - E
ROWS_PT = EROWS // 16
CH = 25
NCHUNK = ROWS_PT // CH
ACC_ROWS = 100008
INIT_PT = N // 16
INIT_CH = INIT_PT // 2


def _sc_body(src_hbm, dst_hbm, p0_hbm, p1_hbm, out_hbm,
             sidx, didx, rows, acc, sem):
    c = lax.axis_index("c")
    s = lax.axis_index("s")

    def run(p_hbm, core):
        r0 = s * INIT_PT
        for k in range(2):
            pltpu.sync_copy(p_hbm.at[pl.ds(r0 + k * INIT_CH, INIT_CH), :],
                            acc.at[pl.ds(r0 + k * INIT_CH, INIT_CH), :])
        plsc.subcore_barrier()

        def chunk(j, carry):
            er = s * ROWS_PT + j * CH
            pltpu.sync_copy(src_hbm.at[pl.ds(er, CH), :], sidx)
            pltpu.sync_copy(dst_hbm.at[pl.ds(er, CH), :], didx)
            pltpu.async_copy(p_hbm.at[sidx], rows, sem).wait()
            pltpu.sync_copy(rows, acc.at[didx], add=True)
            return carry

        lax.fori_loop(0, NCHUNK, chunk, 0)
        plsc.subcore_barrier()
        for k in range(2):
            pltpu.sync_copy(acc.at[pl.ds(r0 + k * INIT_CH, INIT_CH), :],
                            out_hbm.at[core, pl.ds(r0 + k * INIT_CH, INIT_CH), :])

    @pl.when(c == 0)
    def _():
        run(p0_hbm, 0)

    @pl.when(c == 1)
    def _():
        run(p1_hbm, 1)


_sc_seg_sum = functools.partial(
    pl.kernel,
    _sc_body,
    out_type=jax.ShapeDtypeStruct((2, N, HALF), jnp.float32),
    mesh=plsc.VectorSubcoreMesh(core_axis_name="c", subcore_axis_name="s"),
    scratch_types=[
        pltpu.VMEM((CH, 128), jnp.int32),
        pltpu.VMEM((CH, 128), jnp.int32),
        pltpu.VMEM((CH * 128, HALF), jnp.float32),
        pltpu.VMEM_SHARED((ACC_ROWS, HALF), jnp.float32),
        pltpu.SemaphoreType.DMA,
    ],
)()


def _a0_body(x_ref, w1a_ref, w1b_ref, p0_ref, p1_ref):
    x = x_ref[...]
    p0_ref[...] = jnp.dot(x, w1a_ref[...], preferred_element_type=jnp.float32)
    p1_ref[...] = jnp.dot(x, w1b_ref[...], preferred_element_type=jnp.float32)


def _a_body(u_ref, st_ref, g_ref, bb_ref, w1a_ref, w1b_ref, p0_ref, p1_ref):
    st = st_ref[...]
    mu = st[0:1, :] * (1.0 / N)
    var = st[1:2, :] * (1.0 / N) - mu * mu
    scale = lax.rsqrt(var + 1e-5) * g_ref[...]
    h = (u_ref[...] - mu) * scale + bb_ref[...]
    p0_ref[...] = jnp.dot(h, w1a_ref[...], preferred_element_type=jnp.float32)
    p1_ref[...] = jnp.dot(h, w1b_ref[...], preferred_element_type=jnp.float32)


def _b_body(agg0_ref, agg1_ref, b1a_ref, b1b_ref, w2a_ref, w2b_ref, b2_ref,
            u_ref, st_ref):
    i = pl.program_id(0)
    r0 = jnp.maximum(jnp.reshape(agg0_ref[...], (BLK, HALF)) + b1a_ref[...], 0.0)
    r1 = jnp.maximum(jnp.reshape(agg1_ref[...], (BLK, HALF)) + b1b_ref[...], 0.0)
    m = (jnp.dot(r0, w2a_ref[...], preferred_element_type=jnp.float32)
         + jnp.dot(r1, w2b_ref[...], preferred_element_type=jnp.float32)
         + b2_ref[...])
    u = jnp.maximum(m, 0.0)
    u_ref[...] = u

    @pl.when(i == 0)
    def _():
        st_ref[...] = jnp.zeros((2, DIM), jnp.float32)

    s1 = jnp.sum(u, axis=0, keepdims=True)
    s2 = jnp.sum(u * u, axis=0, keepdims=True)
    st_ref[...] += jnp.concatenate([s1, s2], axis=0)


def _f_body(u_ref, st_ref, g_ref, bb_ref, bt_ref, f1w_ref, f1b_ref,
            f2w_ref, f2b_ref, o_ref, pool_acc):
    i = pl.program_id(0)
    st = st_ref[...]
    mu = st[0:1, :] * (1.0 / N)
    var = st[1:2, :] * (1.0 / N) - mu * mu
    scale = lax.rsqrt(var + 1e-5) * g_ref[...]
    h = (u_ref[...] - mu) * scale + bb_ref[...]

    @pl.when(i == 0)
    def _():
        pool_acc[...] = jnp.zeros((N_GRAPHS, DIM), jnp.float32)

    bt = jnp.reshape(bt_ref[...], (1, BLK))
    oh = (lax.broadcasted_iota(jnp.int32, (N_GRAPHS, BLK), 0) == bt
          ).astype(jnp.float32)
    pool_acc[...] += jnp.dot(oh, h, preferred_element_type=jnp.float32)

    @pl.when(i == GRID - 1)
    def _():
        pooled = pool_acc[...]
        z = jnp.maximum(
            jnp.dot(pooled, f1w_ref[...], preferred_element_type=jnp.float32)
            + f1b_ref[...], 0.0)
        z = (jnp.dot(z, f2w_ref[...], preferred_element_type=jnp.float32)
             + f2b_ref[...])
        zm = jnp.max(z, axis=1, keepdims=True)
        lse = jnp.log(jnp.sum(jnp.exp(z - zm), axis=1, keepdims=True)) + zm
        o_ref[...] = z - lse


def _mk_a0():
    return pl.pallas_call(
        _a0_body,
        grid=(GRID,),
        in_specs=[
            pl.BlockSpec((BLK, D_IN), lambda i: (i, 0)),
            pl.BlockSpec((D_IN, HALF), lambda i: (0, 0)),
            pl.BlockSpec((D_IN, HALF), lambda i: (0, 0)),
        ],
        out_specs=[pl.BlockSpec((BLK, HALF), lambda i: (i, 0))] * 2,
        out_shape=[jax.ShapeDtypeStruct((N, HALF), jnp.float32)] * 2,
    )


def _mk_a():
    return pl.pallas_call(
        _a_body,
        grid=(GRID,),
        in_specs=[
            pl.BlockSpec((BLK, DIM), lambda i: (i, 0)),
            pl.BlockSpec((2, DIM), lambda i: (0, 0)),
            pl.BlockSpec((1, DIM), lambda i: (0, 0)),
            pl.BlockSpec((1, DIM), lambda i: (0, 0)),
            pl.BlockSpec((DIM, HALF), lambda i: (0, 0)),
            pl.BlockSpec((DIM, HALF), lambda i: (0, 0)),
        ],
        out_specs=[pl.BlockSpec((BLK, HALF), lambda i: (i, 0))] * 2,
        out_shape=[jax.ShapeDtypeStruct((N, HALF), jnp.float32)] * 2,
    )


def _mk_b():
    return pl.pallas_call(
        _b_body,
        grid=(GRID,),
        in_specs=[
            pl.BlockSpec((1, BLK, HALF), lambda i: (0, i, 0)),
            pl.BlockSpec((1, BLK, HALF), lambda i: (1, i, 0)),
            pl.BlockSpec((1, HALF), lambda i: (0, 0)),
            pl.BlockSpec((1, HALF), lambda i: (0, 0)),
            pl.BlockSpec((HALF, DIM), lambda i: (0, 0)),
            pl.BlockSpec((HALF, DIM), lambda i: (0, 0)),
            pl.BlockSpec((1, DIM), lambda i: (0, 0)),
        ],
        out_specs=[
            pl.BlockSpec((BLK, DIM), lambda i: (i, 0)),
            pl.BlockSpec((2, DIM), lambda i: (0, 0)),
        ],
        out_shape=[
            jax.ShapeDtypeStruct((N, DIM), jnp.float32),
            jax.ShapeDtypeStruct((2, DIM), jnp.float32),
        ],
    )


def _mk_f():
    return pl.pallas_call(
        _f_body,
        grid=(GRID,),
        in_specs=[
            pl.BlockSpec((BLK, DIM), lambda i: (i, 0)),
            pl.BlockSpec((2, DIM), lambda i: (0, 0)),
            pl.BlockSpec((1, DIM), lambda i: (0, 0)),
            pl.BlockSpec((1, DIM), lambda i: (0, 0)),
            pl.BlockSpec((1, 1, BLK), lambda i: (i, 0, 0)),
            pl.BlockSpec((DIM, DIM), lambda i: (0, 0)),
            pl.BlockSpec((1, DIM), lambda i: (0, 0)),
            pl.BlockSpec((DIM, N_CLASSES), lambda i: (0, 0)),
            pl.BlockSpec((1, N_CLASSES), lambda i: (0, 0)),
        ],
        out_specs=pl.BlockSpec((N_GRAPHS, N_CLASSES), lambda i: (0, 0)),
        out_shape=jax.ShapeDtypeStruct((N_GRAPHS, N_CLASSES), jnp.float32),
        scratch_shapes=[pltpu.VMEM((N_GRAPHS, DIM), jnp.float32)],
    )


def kernel(x, edge_index, batch, params):
    src = edge_index[0]
    dst = edge_index[1]
    srcp = jnp.concatenate(
        [src, jnp.zeros((EPAD,), jnp.int32)]).reshape(EROWS, 128)
    dstp = jnp.concatenate(
        [dst, jnp.full((EPAD,), N, jnp.int32)]).reshape(EROWS, 128)
    bt3 = batch.reshape(GRID, 1, BLK)

    a0 = _mk_a0()
    a_call = _mk_a()
    b_call = _mk_b()
    f_call = _mk_f()

    c0 = params["convs"][0]
    p0, p1 = a0(x, c0["w1"][:, :HALF], c0["w1"][:, HALF:])
    u = st = None
    for l in range(5):
        c = params["convs"][l]
        if l > 0:
            bn = params["bns"][l - 1]
            p0, p1 = a_call(u, st, bn["g"].reshape(1, DIM),
                            bn["b"].reshape(1, DIM),
                            c["w1"][:, :HALF], c["w1"][:, HALF:])
        agg = _sc_seg_sum(srcp, dstp, p0, p1)
        u, st = b_call(agg, agg,
                       c["b1"][:HALF].reshape(1, HALF),
                       c["b1"][HALF:].reshape(1, HALF),
                       c["w2"][:HALF, :], c["w2"][HALF:, :],
                       c["b2"].reshape(1, DIM))
    bn = params["bns"][4]
    return f_call(u, st, bn["g"].reshape(1, DIM), bn["b"].reshape(1, DIM),
                  bt3, params["fc1_w"], params["fc1_b"].reshape(1, DIM),
                  params["fc2_w"], params["fc2_b"].reshape(1, N_CLASSES))


# speedup vs baseline: 6.8464x; 6.8464x over previous
import functools

import jax
import jax.numpy as jnp
from jax import lax
from jax.experimental import pallas as pl
from jax.experimental.pallas import tpu as pltpu
from jax.experimental.pallas import tpu_sc as plsc

N = 100000
E = 1600000
D_IN = 128
DIM = 32
HALF = 16
N_GRAPHS = 128
N_CLASSES = 10

BLK = 2000
GRID = N // BLK

CHE = 1024
NCHUNK = -(-E // (16 * CHE))
EPT = NCHUNK * CHE
EPAD = 16 * EPT - E
ACC_ROWS = 100008
INIT_PT = 6256
INIT_LAST = N - 15 * INIT_PT


def _sc_pass(src_hbm, dst_hbm, p_hbm, out_hbm, out_idx,
             sidx, didx, rows, acc, sem, s):
    r0 = s * INIT_PT

    @pl.when(s < 15)
    def _():
        pltpu.sync_copy(p_hbm.at[pl.ds(r0, INIT_PT), :],
                        acc.at[pl.ds(r0, INIT_PT), :])

    @pl.when(s == 15)
    def _():
        pltpu.sync_copy(p_hbm.at[pl.ds(15 * INIT_PT, INIT_LAST), :],
                        acc.at[pl.ds(15 * INIT_PT, INIT_LAST), :])

    plsc.subcore_barrier()

    def chunk(j, carry):
        e0 = s * EPT + j * CHE
        pltpu.sync_copy(src_hbm.at[pl.ds(e0, CHE)], sidx)
        pltpu.sync_copy(dst_hbm.at[pl.ds(e0, CHE)], didx)
        pltpu.async_copy(p_hbm.at[sidx], rows, sem).wait()
        pltpu.sync_copy(rows, acc.at[didx], add=True)
        return carry

    lax.fori_loop(0, NCHUNK, chunk, 0)
    plsc.subcore_barrier()

    @pl.when(s < 15)
    def _():
        pltpu.sync_copy(acc.at[pl.ds(r0, INIT_PT), :],
                        out_hbm.at[out_idx, pl.ds(r0, INIT_PT), :])

    @pl.when(s == 15)
    def _():
        pltpu.sync_copy(acc.at[pl.ds(15 * INIT_PT, INIT_LAST), :],
                        out_hbm.at[out_idx, pl.ds(15 * INIT_PT, INIT_LAST), :])

    plsc.subcore_barrier()


def _sc_body2(src_hbm, dst_hbm, p0_hbm, p1_hbm, out_hbm,
              sidx, didx, rows, acc, sem):
    c = lax.axis_index("c")
    s = lax.axis_index("s")

    @pl.when(c == 0)
    def _():
        _sc_pass(src_hbm, dst_hbm, p0_hbm, out_hbm, 0,
                 sidx, didx, rows, acc, sem, s)

    @pl.when(c == 1)
    def _():
        _sc_pass(src_hbm, dst_hbm, p1_hbm, out_hbm, 1,
                 sidx, didx, rows, acc, sem, s)


def _sc_body8(src_hbm, dst_hbm, x0, x1, x2, x3, x4, x5, x6, x7, out_hbm,
              sidx, didx, rows, acc, sem):
    c = lax.axis_index("c")
    s = lax.axis_index("s")
    slabs = (x0, x1, x2, x3, x4, x5, x6, x7)
    for t in range(4):
        @pl.when(c == 0)
        def _(t=t):
            _sc_pass(src_hbm, dst_hbm, slabs[2 * t], out_hbm, 2 * t,
                     sidx, didx, rows, acc, sem, s)

        @pl.when(c == 1)
        def _(t=t):
            _sc_pass(src_hbm, dst_hbm, slabs[2 * t + 1], out_hbm, 2 * t + 1,
                     sidx, didx, rows, acc, sem, s)


_SC_SCRATCH = [
    pltpu.VMEM((CHE,), jnp.int32),
    pltpu.VMEM((CHE,), jnp.int32),
    pltpu.VMEM((CHE, HALF), jnp.float32),
    pltpu.VMEM_SHARED((ACC_ROWS, HALF), jnp.float32),
    pltpu.SemaphoreType.DMA,
]


@functools.cache
def _mk_sc2():
    return pl.kernel(
        _sc_body2,
        out_type=jax.ShapeDtypeStruct((2, N, HALF), jnp.float32),
        mesh=plsc.VectorSubcoreMesh(core_axis_name="c", subcore_axis_name="s",
                                    num_cores=2, num_subcores=16),
        scratch_types=list(_SC_SCRATCH),
        compiler_params=pltpu.CompilerParams(use_tc_tiling_on_sc=False),
    )


@functools.cache
def _mk_sc8():
    return pl.kernel(
        _sc_body8,
        out_type=jax.ShapeDtypeStruct((8, N, HALF), jnp.float32),
        mesh=plsc.VectorSubcoreMesh(core_axis_name="c", subcore_axis_name="s",
                                    num_cores=2, num_subcores=16),
        scratch_types=list(_SC_SCRATCH),
        compiler_params=pltpu.CompilerParams(use_tc_tiling_on_sc=False),
    )


def _sc_seg_sum(srcp, dstp, h0, h1):
    return _mk_sc2()(srcp, dstp, h0, h1)


def _sc_seg_sum_x(srcp, dstp, slabs):
    return _mk_sc8()(srcp, dstp, *slabs)


def _bn_body(u_ref, st_ref, g_ref, bb_ref, h0_ref, h1_ref):
    st = st_ref[...]
    mu = st[0:1, :] * (1.0 / N)
    var = st[1:2, :] * (1.0 / N) - mu * mu
    scale = lax.rsqrt(var + 1e-5) * g_ref[...]
    h = (u_ref[...] - mu) * scale + bb_ref[...]
    h0_ref[...] = h[:, :HALF]
    h1_ref[...] = h[:, HALF:]


def _mlp_tail(m, w2_ref, b2_ref, u_ref, st_ref, i):
    r = jnp.maximum(m, 0.0)
    u = jnp.maximum(
        jnp.dot(r, w2_ref[...], preferred_element_type=jnp.float32)
        + b2_ref[...], 0.0)
    u_ref[...] = u

    @pl.when(i == 0)
    def _():
        st_ref[...] = jnp.zeros((2, DIM), jnp.float32)

    s1 = jnp.sum(u, axis=0, keepdims=True)
    s2 = jnp.sum(u * u, axis=0, keepdims=True)
    st_ref[...] += jnp.concatenate([s1, s2], axis=0)


def _b0_body(a0, a1, a2, a3, a4, a5, a6, a7, w1_ref, b1_ref, w2_ref, b2_ref,
             u_ref, st_ref):
    i = pl.program_id(0)
    hagg = jnp.concatenate(
        [jnp.reshape(a[...], (BLK, HALF)) for a in
         (a0, a1, a2, a3, a4, a5, a6, a7)], axis=1)
    m = (jnp.dot(hagg, w1_ref[...], preferred_element_type=jnp.float32)
         + b1_ref[...])
    _mlp_tail(m, w2_ref, b2_ref, u_ref, st_ref, i)


def _b_body(a0_ref, a1_ref, w1_ref, b1_ref, w2_ref, b2_ref, u_ref, st_ref):
    i = pl.program_id(0)
    hagg = jnp.concatenate([jnp.reshape(a0_ref[...], (BLK, HALF)),
                            jnp.reshape(a1_ref[...], (BLK, HALF))], axis=1)
    m = (jnp.dot(hagg, w1_ref[...], preferred_element_type=jnp.float32)
         + b1_ref[...])
    _mlp_tail(m, w2_ref, b2_ref, u_ref, st_ref, i)


def _f_body(u_ref, st_ref, g_ref, bb_ref, bt_ref, f1w_ref, f1b_ref,
            f2w_ref, f2b_ref, o_ref, pool_acc):
    i = pl.program_id(0)
    st = st_ref[...]
    mu = st[0:1, :] * (1.0 / N)
    var = st[1:2, :] * (1.0 / N) - mu * mu
    scale = lax.rsqrt(var + 1e-5) * g_ref[...]
    h = (u_ref[...] - mu) * scale + bb_ref[...]

    @pl.when(i == 0)
    def _():
        pool_acc[...] = jnp.zeros((N_GRAPHS, DIM), jnp.float32)

    bt = jnp.reshape(bt_ref[...], (1, BLK))
    oh = (lax.broadcasted_iota(jnp.int32, (N_GRAPHS, BLK), 0) == bt
          ).astype(jnp.float32)
    pool_acc[...] += jnp.dot(oh, h, preferred_element_type=jnp.float32,
                             precision=lax.Precision.HIGHEST)

    @pl.when(i == GRID - 1)
    def _():
        pooled = pool_acc[...]
        z = jnp.maximum(
            jnp.dot(pooled, f1w_ref[...], preferred_element_type=jnp.float32)
            + f1b_ref[...], 0.0)
        z = (jnp.dot(z, f2w_ref[...], preferred_element_type=jnp.float32)
             + f2b_ref[...])
        zm = jnp.max(z, axis=1, keepdims=True)
        lse = jnp.log(jnp.sum(jnp.exp(z - zm), axis=1, keepdims=True)) + zm
        o_ref[...] = z - lse


_W = pl.BlockSpec((2, DIM), lambda i: (0, 0))
_V = pl.BlockSpec((1, DIM), lambda i: (0, 0))


def _slab_spec(k):
    return pl.BlockSpec((1, BLK, HALF), lambda i, k=k: (k, i, 0))


def _mk_bn():
    return pl.pallas_call(
        _bn_body,
        grid=(GRID,),
        in_specs=[pl.BlockSpec((BLK, DIM), lambda i: (i, 0)), _W, _V, _V],
        out_specs=[pl.BlockSpec((BLK, HALF), lambda i: (i, 0))] * 2,
        out_shape=[jax.ShapeDtypeStruct((N, HALF), jnp.float32)] * 2,
    )


def _mk_b0():
    return pl.pallas_call(
        _b0_body,
        grid=(GRID,),
        in_specs=[_slab_spec(k) for k in range(8)] + [
            pl.BlockSpec((D_IN, DIM), lambda i: (0, 0)),
            _V,
            pl.BlockSpec((DIM, DIM), lambda i: (0, 0)),
            _V,
        ],
        out_specs=[pl.BlockSpec((BLK, DIM), lambda i: (i, 0)), _W],
        out_shape=[jax.ShapeDtypeStruct((N, DIM), jnp.float32),
                   jax.ShapeDtypeStruct((2, DIM), jnp.float32)],
    )


def _mk_b():
    return pl.pallas_call(
        _b_body,
        grid=(GRID,),
        in_specs=[_slab_spec(0), _slab_spec(1),
                  pl.BlockSpec((DIM, DIM), lambda i: (0, 0)),
                  _V,
                  pl.BlockSpec((DIM, DIM), lambda i: (0, 0)),
                  _V],
        out_specs=[pl.BlockSpec((BLK, DIM), lambda i: (i, 0)), _W],
        out_shape=[jax.ShapeDtypeStruct((N, DIM), jnp.float32),
                   jax.ShapeDtypeStruct((2, DIM), jnp.float32)],
    )


def _mk_f():
    return pl.pallas_call(
        _f_body,
        grid=(GRID,),
        in_specs=[
            pl.BlockSpec((BLK, DIM), lambda i: (i, 0)),
            _W, _V, _V,
            pl.BlockSpec((1, 1, BLK), lambda i: (i, 0, 0)),
            pl.BlockSpec((DIM, DIM), lambda i: (0, 0)),
            _V,
            pl.BlockSpec((DIM, N_CLASSES), lambda i: (0, 0)),
            pl.BlockSpec((1, N_CLASSES), lambda i: (0, 0)),
        ],
        out_specs=pl.BlockSpec((N_GRAPHS, N_CLASSES), lambda i: (0, 0)),
        out_shape=jax.ShapeDtypeStruct((N_GRAPHS, N_CLASSES), jnp.float32),
        scratch_shapes=[pltpu.VMEM((N_GRAPHS, DIM), jnp.float32)],
    )


def kernel(x, edge_index, batch, params):
    src = edge_index[0]
    dst = edge_index[1]
    pad_src = (jnp.arange(EPAD, dtype=jnp.int32) * 16) % N
    pad_dst = N + (jnp.arange(EPAD, dtype=jnp.int32) % (ACC_ROWS - N))
    srcp = jnp.concatenate([src, pad_src])
    dstp = jnp.concatenate([dst, pad_dst])
    bt3 = batch.reshape(GRID, 1, BLK)
    xs = tuple(x[:, k * HALF:(k + 1) * HALF] for k in range(8))

    bn_call = _mk_bn()
    b0_call = _mk_b0()
    b_call = _mk_b()
    f_call = _mk_f()

    u = st = None
    for l in range(5):
        c = params["convs"][l]
        if l == 0:
            aggx = _sc_seg_sum_x(srcp, dstp, xs)
            u, st = b0_call(*([aggx] * 8), c["w1"],
                            c["b1"].reshape(1, DIM), c["w2"],
                            c["b2"].reshape(1, DIM))
        else:
            bn = params["bns"][l - 1]
            h0, h1 = bn_call(u, st, bn["g"].reshape(1, DIM),
                             bn["b"].reshape(1, DIM))
            agg = _sc_seg_sum(srcp, dstp, h0, h1)
            u, st = b_call(agg, agg, c["w1"], c["b1"].reshape(1, DIM),
                           c["w2"], c["b2"].reshape(1, DIM))
    bn = params["bns"][4]
    return f_call(u, st, bn["g"].reshape(1, DIM), bn["b"].reshape(1, DIM),
                  bt3, params["fc1_w"], params["fc1_b"].reshape(1, DIM),
                  params["fc2_w"], params["fc2_b"].reshape(1, N_CLASSES))
